# baseline (speedup 1.0000x reference)
import functools

import jax
import jax.numpy as jnp
from jax import lax
from jax.experimental import pallas as pl
from jax.experimental.pallas import tpu as pltpu
from jax.experimental.pallas import tpu_sc as plsc


def _matmul_body(x_ref, w_ref, y_ref):
    y_ref[...] = jnp.dot(x_ref[...], w_ref[0], preferred_element_type=jnp.float32)


def _relation_matmuls(X, W, bm):
    N, D = X.shape
    R, _, U = W.shape
    nm = N // bm
    return pl.pallas_call(
        _matmul_body,
        grid=(R, nm),
        in_specs=[
            pl.BlockSpec((bm, D), lambda r, m: (m, 0)),
            pl.BlockSpec((1, D, U), lambda r, m: (r, 0, 0)),
        ],
        out_specs=pl.BlockSpec((bm, U), lambda r, m: (r * nm + m, 0)),
        out_shape=jax.ShapeDtypeStruct((R * N, U), jnp.float32),
    )(X, W)


def _combine_body(p_ref, x_ref, w0_ref, b_ref, o_ref):
    z = jnp.dot(x_ref[...], w0_ref[...], preferred_element_type=jnp.float32)
    acc = p_ref[0] + p_ref[1] + z + b_ref[...]
    o_ref[...] = jnp.maximum(acc, 0.0)


def _combine(P, X, W_0, b, bm):
    N, D = X.shape
    U = W_0.shape[1]
    nm = N // bm
    return pl.pallas_call(
        _combine_body,
        grid=(nm,),
        in_specs=[
            pl.BlockSpec((2, bm, U), lambda m: (0, m, 0)),
            pl.BlockSpec((bm, D), lambda m: (m, 0)),
            pl.BlockSpec((D, U), lambda m: (0, 0)),
            pl.BlockSpec((1, U), lambda m: (0, 0)),
        ],
        out_specs=pl.BlockSpec((bm, U), lambda m: (m, 0)),
        out_shape=jax.ShapeDtypeStruct((N, U), jnp.float32),
    )(P, X, W_0, b.reshape(1, U))


_K = 16


def _make_scatter(N_acc, U, n_sc, n_sub, chunk, nb, rpt):
    mesh = plsc.VectorSubcoreMesh(core_axis_name="c", subcore_axis_name="s")
    assert nb % 2 == 0
    kc = _K * chunk

    @functools.partial(
        pl.kernel,
        out_type=jax.ShapeDtypeStruct((n_sc, N_acc, U), jnp.float32),
        mesh=mesh,
        scratch_types=[
            pltpu.VMEM((kc,), jnp.int32),
            pltpu.VMEM((kc,), jnp.int32),
            pltpu.VMEM((_K, chunk), jnp.int32),
            pltpu.VMEM((_K, chunk), jnp.int32),
            pltpu.VMEM((chunk, U), jnp.float32),
            pltpu.VMEM((chunk, U), jnp.float32),
            pltpu.VMEM_SHARED((N_acc, U), jnp.float32),
            pltpu.SemaphoreType.DMA,
            pltpu.SemaphoreType.DMA,
            pltpu.SemaphoreType.DMA,
            pltpu.SemaphoreType.DMA,
        ],
    )
    def scatter_kernel(y_hbm, src_hbm, dst_hbm, zeros_hbm, out_hbm,
                       sidx0, sidx1, didx0, didx1, rows0, rows1, acc,
                       gsem0, gsem1, isem0, isem1):
        sidx = (sidx0, sidx1)
        didx = (didx0, didx1)
        rows = (rows0, rows1)
        gsem = (gsem0, gsem1)
        isem = (isem0, isem1)
        c = lax.axis_index("c")
        s = lax.axis_index("s")
        wid = s * n_sc + c
        sbase = wid * (nb + 2) * kc
        dbase = wid * (nb + 2) * _K

        def fetch_descs(m, sl):
            return (
                pltpu.make_async_copy(
                    src_hbm.at[pl.ds(sbase + m * kc, kc)], sidx[sl], isem[sl]),
                pltpu.make_async_copy(
                    dst_hbm.at[pl.ds(dbase + m * _K, _K)], didx[sl], isem[sl]),
            )

        def gather_desc(isl, jj, rsl):
            return pltpu.make_async_copy(
                y_hbm.at[sidx[isl].at[pl.ds(jj * chunk, chunk)]],
                rows[rsl], gsem[rsl])

        pltpu.sync_copy(zeros_hbm, acc.at[pl.ds(s * rpt, rpt)])
        for d in fetch_descs(0, 0):
            d.start()
        for d in fetch_descs(0, 0):
            d.wait()
        for d in fetch_descs(1, 1):
            d.start()
        plsc.subcore_barrier()
        gather_desc(0, 0, 0).start()

        def two_batches(m2, carry):
            for mb in range(2):
                m = m2 * 2 + mb
                for jj in range(_K):
                    rsl = jj % 2
                    gather_desc(mb, jj, rsl).wait()
                    if jj < _K - 1:
                        gather_desc(mb, jj + 1, 1 - rsl).start()
                    else:
                        for d in fetch_descs(m + 1, 1 - mb):
                            d.wait()
                        gather_desc(1 - mb, 0, 1 - rsl).start()
                    pltpu.sync_copy(rows[rsl], acc.at[didx[mb].at[jj]],
                                    add=True)
                for d in fetch_descs(m + 2, mb):
                    d.start()
            return carry

        lax.fori_loop(0, nb // 2, two_batches, 0)
        for d in fetch_descs(nb + 1, 1):
            d.wait()
        gather_desc(0, 0, 0).wait()
        plsc.subcore_barrier()
        pltpu.sync_copy(acc.at[pl.ds(s * rpt, rpt)],
                        out_hbm.at[c, pl.ds(s * rpt, rpt)])

    return scatter_kernel


def kernel(X, ref_a, ref_b, W, W_0, b):
    N, D = X.shape
    R, _, U = W.shape
    E = ref_a.shape[1]

    info = plsc.get_sparse_core_info()
    n_sc, n_sub = info.num_cores, info.num_subcores
    nw = n_sc * n_sub
    chunk = 128
    rpt = -(-N // n_sub)
    rpt = ((rpt + 7) // 8) * 8
    N_acc = n_sub * rpt
    if N_acc == N:
        rpt += 8
        N_acc = n_sub * rpt
    dst_pad = N

    offs = (jnp.arange(R, dtype=jnp.int32) * N)[:, None]
    srcs = jnp.concatenate([(ref_b + offs).reshape(-1), (ref_a + offs).reshape(-1)])
    dsts = jnp.concatenate([ref_a.reshape(-1), ref_b.reshape(-1)])
    total = 2 * R * E
    kc = _K * chunk
    cpw = -(-total // (nw * chunk))
    cpw = -(-cpw // (2 * _K)) * (2 * _K)
    nb = cpw // _K
    pad = nw * cpw * chunk - total
    if pad:
        srcs = jnp.concatenate([srcs, jnp.zeros((pad,), jnp.int32)])
        dsts = jnp.concatenate([dsts, jnp.full((pad,), dst_pad, jnp.int32)])
    srcs = jnp.concatenate(
        [srcs.reshape(nw, nb * kc),
         jnp.zeros((nw, 2 * kc), jnp.int32)], axis=1).reshape(-1)
    dsts = jnp.concatenate(
        [dsts.reshape(nw, nb * _K, chunk),
         jnp.full((nw, 2 * _K, chunk), dst_pad, jnp.int32)],
        axis=1).reshape(-1, chunk)

    bm = 400
    Y = _relation_matmuls(X, W, bm)

    zeros_hbm = jnp.zeros((rpt, U), jnp.float32)
    P = _make_scatter(N_acc, U, n_sc, n_sub, chunk, nb, rpt)(
        Y, srcs, dsts, zeros_hbm)

    return _combine(P, X, W_0, b, bm)


# speedup vs baseline: 1.0636x; 1.0636x over previous
import functools

import jax
import jax.numpy as jnp
from jax import lax
from jax.experimental import pallas as pl
from jax.experimental.pallas import tpu as pltpu
from jax.experimental.pallas import tpu_sc as plsc


def _combine_body(p_ref, x_ref, w_ref, w0_ref, b_ref, o_ref):
    z = jnp.dot(p_ref[0], w_ref[0], preferred_element_type=jnp.float32)
    z += jnp.dot(p_ref[1], w_ref[1], preferred_element_type=jnp.float32)
    z += jnp.dot(x_ref[...], w0_ref[...], preferred_element_type=jnp.float32)
    o_ref[...] = jnp.maximum(z + b_ref[...], 0.0)


def _combine(P, Xp, W, W_0, b, bm):
    n_sc, Nr, U = P.shape
    D = Xp.shape[1]
    nm = Nr // bm
    return pl.pallas_call(
        _combine_body,
        grid=(nm,),
        in_specs=[
            pl.BlockSpec((n_sc, bm, U), lambda m: (0, m, 0)),
            pl.BlockSpec((bm, D), lambda m: (m, 0)),
            pl.BlockSpec((n_sc, D, U), lambda m: (0, 0, 0)),
            pl.BlockSpec((D, U), lambda m: (0, 0)),
            pl.BlockSpec((1, U), lambda m: (0, 0)),
        ],
        out_specs=pl.BlockSpec((bm, U), lambda m: (m, 0)),
        out_shape=jax.ShapeDtypeStruct((Nr, U), jnp.float32),
    )(P, Xp, W, W_0, b.reshape(1, U))


_K = 16


def _make_scatter(Nr, U, n_sc, n_sub, chunk, nb, rpt):
    mesh = plsc.VectorSubcoreMesh(core_axis_name="c", subcore_axis_name="s")
    assert nb % 2 == 0
    kc = _K * chunk

    @functools.partial(
        pl.kernel,
        out_type=jax.ShapeDtypeStruct((n_sc, Nr, U), jnp.float32),
        mesh=mesh,
        scratch_types=[
            pltpu.VMEM((kc,), jnp.int32),
            pltpu.VMEM((kc,), jnp.int32),
            pltpu.VMEM((_K, chunk), jnp.int32),
            pltpu.VMEM((_K, chunk), jnp.int32),
            pltpu.VMEM((chunk, U), jnp.float32),
            pltpu.VMEM((chunk, U), jnp.float32),
            pltpu.VMEM_SHARED((Nr, U), jnp.float32),
            pltpu.SemaphoreType.DMA,
            pltpu.SemaphoreType.DMA,
            pltpu.SemaphoreType.DMA,
            pltpu.SemaphoreType.DMA,
        ],
    )
    def scatter_kernel(x_hbm, src_hbm, dst_hbm, zeros_hbm, out_hbm,
                       sidx0, sidx1, didx0, didx1, rows0, rows1, acc,
                       gsem0, gsem1, isem0, isem1):
        sidx = (sidx0, sidx1)
        didx = (didx0, didx1)
        rows = (rows0, rows1)
        gsem = (gsem0, gsem1)
        isem = (isem0, isem1)
        c = lax.axis_index("c")
        s = lax.axis_index("s")
        wid = s * n_sc + c
        sbase = wid * (nb + 2) * kc
        dbase = wid * (nb + 2) * _K

        def fetch_descs(m, sl):
            return (
                pltpu.make_async_copy(
                    src_hbm.at[pl.ds(sbase + m * kc, kc)], sidx[sl], isem[sl]),
                pltpu.make_async_copy(
                    dst_hbm.at[pl.ds(dbase + m * _K, _K)], didx[sl], isem[sl]),
            )

        def gather_desc(isl, jj, rsl):
            return pltpu.make_async_copy(
                x_hbm.at[sidx[isl].at[pl.ds(jj * chunk, chunk)]],
                rows[rsl], gsem[rsl])

        pltpu.sync_copy(zeros_hbm, acc.at[pl.ds(s * rpt, rpt)])
        for d in fetch_descs(0, 0):
            d.start()
        for d in fetch_descs(0, 0):
            d.wait()
        for d in fetch_descs(1, 1):
            d.start()
        plsc.subcore_barrier()
        gather_desc(0, 0, 0).start()

        def two_batches(m2, carry):
            for mb in range(2):
                m = m2 * 2 + mb
                for jj in range(_K):
                    rsl = jj % 2
                    gather_desc(mb, jj, rsl).wait()
                    if jj < _K - 1:
                        gather_desc(mb, jj + 1, 1 - rsl).start()
                    else:
                        for d in fetch_descs(m + 1, 1 - mb):
                            d.wait()
                        gather_desc(1 - mb, 0, 1 - rsl).start()
                    pltpu.sync_copy(rows[rsl], acc.at[didx[mb].at[jj]],
                                    add=True)
                for d in fetch_descs(m + 2, mb):
                    d.start()
            return carry

        lax.fori_loop(0, nb // 2, two_batches, 0)
        for d in fetch_descs(nb + 1, 1):
            d.wait()
        gather_desc(0, 0, 0).wait()
        plsc.subcore_barrier()
        pltpu.sync_copy(acc.at[pl.ds(s * rpt, rpt)],
                        out_hbm.at[c, pl.ds(s * rpt, rpt)])

    return scatter_kernel


def kernel(X, ref_a, ref_b, W, W_0, b):
    N, D = X.shape
    R, _, U = W.shape
    E = ref_a.shape[1]

    info = plsc.get_sparse_core_info()
    n_sc, n_sub = info.num_cores, info.num_subcores
    chunk = 128
    bm = 512
    Nr = -(-(N + 1) // bm) * bm
    rpt = Nr // n_sub
    assert rpt % 8 == 0 and Nr % bm == 0
    dst_pad = N

    srcs = jnp.concatenate([ref_b, ref_a], axis=1)
    dsts = jnp.concatenate([ref_a, ref_b], axis=1)
    per_rel = 2 * E
    kc = _K * chunk
    cpw = -(-per_rel // (n_sub * chunk))
    cpw = -(-cpw // (2 * _K)) * (2 * _K)
    nb = cpw // _K
    pad = n_sub * cpw * chunk - per_rel
    if pad:
        srcs = jnp.concatenate(
            [srcs, jnp.zeros((R, pad), jnp.int32)], axis=1)
        dsts = jnp.concatenate(
            [dsts, jnp.full((R, pad), dst_pad, jnp.int32)], axis=1)
    srcs = jnp.concatenate(
        [srcs.reshape(n_sc, n_sub, nb * kc),
         jnp.zeros((n_sc, n_sub, 2 * kc), jnp.int32)],
        axis=2).transpose(1, 0, 2).reshape(-1)
    dsts = jnp.concatenate(
        [dsts.reshape(n_sc, n_sub, nb * _K, chunk),
         jnp.full((n_sc, n_sub, 2 * _K, chunk), dst_pad, jnp.int32)],
        axis=2).transpose(1, 0, 2, 3).reshape(-1, chunk)

    zeros_hbm = jnp.zeros((rpt, U), jnp.float32)
    P = _make_scatter(Nr, U, n_sc, n_sub, chunk, nb, rpt)(
        X, srcs, dsts, zeros_hbm)

    Xp = jnp.concatenate([X, jnp.zeros((Nr - N, D), jnp.float32)])
    out = _combine(P, Xp, W, W_0, b, bm)
    return out[:N]
